# baseline (speedup 1.0000x reference)
import jax
import jax.numpy as jnp
from jax.experimental import pallas as pl
from jax.experimental.pallas import tpu as pltpu

_B, _T, _D = 4, 4096, 1024
_A0, _A1 = 64, 64

_RPB = 8
_TB = _RPB * _A1
_NBLK = _A0 // _RPB


def _body(w0_ref, w1_ref, out_ref, scr_ref, sem_ref):
    k = pl.program_id(0)
    slot = jax.lax.rem(k, 2)

    @pl.when(k >= 2)
    def _():
        for b in range(_B):
            pltpu.make_async_copy(
                scr_ref.at[slot],
                out_ref.at[b, pl.ds((k - 2) * _TB, _TB), :],
                sem_ref.at[slot, b],
            ).wait()

    rows = w0_ref[0, :, 0, :]
    tile = w1_ref[0, 0, :, :]
    s = rows[:, None, :] + tile[None, :, :]
    scr_ref[slot] = s.reshape(_TB, _D)

    for b in range(_B):
        pltpu.make_async_copy(
            scr_ref.at[slot],
            out_ref.at[b, pl.ds(k * _TB, _TB), :],
            sem_ref.at[slot, b],
        ).start()

    @pl.when(k == _NBLK - 1)
    def _():
        for kk in (k - 1, k):
            sl = jax.lax.rem(kk, 2)
            for b in range(_B):
                pltpu.make_async_copy(
                    scr_ref.at[sl],
                    out_ref.at[b, pl.ds(kk * _TB, _TB), :],
                    sem_ref.at[sl, b],
                ).wait()


def kernel(x, w0, w1):
    del x
    out = pl.pallas_call(
        _body,
        grid=(_NBLK,),
        in_specs=[
            pl.BlockSpec((1, _RPB, 1, _D), lambda k: (0, k, 0, 0)),
            pl.BlockSpec((1, 1, _A1, _D), lambda k: (0, 0, 0, 0)),
        ],
        out_specs=pl.BlockSpec(memory_space=pltpu.MemorySpace.HBM),
        out_shape=jax.ShapeDtypeStruct((_B, _T, _D), jnp.float32),
        scratch_shapes=[
            pltpu.VMEM((2, _TB, _D), jnp.float32),
            pltpu.SemaphoreType.DMA((2, _B)),
        ],
    )(w0, w1)
    return out


# speedup vs baseline: 1.0180x; 1.0180x over previous
import jax
import jax.numpy as jnp
from jax.experimental import pallas as pl
from jax.experimental.pallas import tpu as pltpu

_B, _T, _D = 4, 4096, 1024
_A0, _A1 = 64, 64

_RPB = 16
_TB = _RPB * _A1
_NBLK = _A0 // _RPB


def _body(w0_ref, w1_ref, out_ref, scr_ref, sem_ref):
    k = pl.program_id(0)
    slot = jax.lax.rem(k, 2)

    @pl.when(k >= 2)
    def _():
        for b in range(_B):
            pltpu.make_async_copy(
                scr_ref.at[slot],
                out_ref.at[b, pl.ds((k - 2) * _TB, _TB), :],
                sem_ref.at[slot, b],
            ).wait()

    rows = w0_ref[0, :, 0, :]
    tile = w1_ref[0, 0, :, :]
    s = rows[:, None, :] + tile[None, :, :]
    scr_ref[slot] = s.reshape(_TB, _D)

    for b in range(_B):
        pltpu.make_async_copy(
            scr_ref.at[slot],
            out_ref.at[b, pl.ds(k * _TB, _TB), :],
            sem_ref.at[slot, b],
        ).start()

    @pl.when(k == _NBLK - 1)
    def _():
        for kk in (k - 1, k):
            sl = jax.lax.rem(kk, 2)
            for b in range(_B):
                pltpu.make_async_copy(
                    scr_ref.at[sl],
                    out_ref.at[b, pl.ds(kk * _TB, _TB), :],
                    sem_ref.at[sl, b],
                ).wait()


def kernel(x, w0, w1):
    del x
    out = pl.pallas_call(
        _body,
        grid=(_NBLK,),
        in_specs=[
            pl.BlockSpec((1, _RPB, 1, _D), lambda k: (0, k, 0, 0)),
            pl.BlockSpec((1, 1, _A1, _D), lambda k: (0, 0, 0, 0)),
        ],
        out_specs=pl.BlockSpec(memory_space=pltpu.MemorySpace.HBM),
        out_shape=jax.ShapeDtypeStruct((_B, _T, _D), jnp.float32),
        scratch_shapes=[
            pltpu.VMEM((2, _TB, _D), jnp.float32),
            pltpu.SemaphoreType.DMA((2, _B)),
        ],
    )(w0, w1)
    return out


# speedup vs baseline: 1.0303x; 1.0121x over previous
import jax
import jax.numpy as jnp
from jax.experimental import pallas as pl
from jax.experimental.pallas import tpu as pltpu

_B, _T, _D = 4, 4096, 1024
_A0, _A1 = 64, 64

_RPB = 4
_TB = _RPB * _A1
_NBLK = _A0 // _RPB


def _body(w0_ref, w1_ref, out_ref, scr_ref, sem_ref):
    k = pl.program_id(0)
    slot = jax.lax.rem(k, 2)

    @pl.when(k >= 2)
    def _():
        for b in range(_B):
            pltpu.make_async_copy(
                scr_ref.at[slot],
                out_ref.at[b, pl.ds((k - 2) * _TB, _TB), :],
                sem_ref.at[slot, b],
            ).wait()

    rows = w0_ref[0, :, 0, :]
    tile = w1_ref[0, 0, :, :]
    s = rows[:, None, :] + tile[None, :, :]
    scr_ref[slot] = s.reshape(_TB, _D)

    for b in range(_B):
        pltpu.make_async_copy(
            scr_ref.at[slot],
            out_ref.at[b, pl.ds(k * _TB, _TB), :],
            sem_ref.at[slot, b],
        ).start()

    @pl.when(k == _NBLK - 1)
    def _():
        for kk in (k - 1, k):
            sl = jax.lax.rem(kk, 2)
            for b in range(_B):
                pltpu.make_async_copy(
                    scr_ref.at[sl],
                    out_ref.at[b, pl.ds(kk * _TB, _TB), :],
                    sem_ref.at[sl, b],
                ).wait()


def kernel(x, w0, w1):
    del x
    out = pl.pallas_call(
        _body,
        grid=(_NBLK,),
        in_specs=[
            pl.BlockSpec((1, _RPB, 1, _D), lambda k: (0, k, 0, 0)),
            pl.BlockSpec((1, 1, _A1, _D), lambda k: (0, 0, 0, 0)),
        ],
        out_specs=pl.BlockSpec(memory_space=pltpu.MemorySpace.HBM),
        out_shape=jax.ShapeDtypeStruct((_B, _T, _D), jnp.float32),
        scratch_shapes=[
            pltpu.VMEM((2, _TB, _D), jnp.float32),
            pltpu.SemaphoreType.DMA((2, _B)),
        ],
    )(w0, w1)
    return out
